# baseline (speedup 1.0000x reference)
import functools
import math

import jax
import jax.numpy as jnp
from jax import lax
from jax.experimental import pallas as pl
from jax.experimental.pallas import tpu as pltpu
from jax.experimental.pallas import tpu_sc as plsc

D = 64
SCALE = math.sqrt(D)

_NC = 2
_NS = 16
_NW = _NC * _NS

SB = 128
TB = 2


@functools.partial(jax.jit, static_argnames=("n_seq", "n_tok"))
def _lookup_t_major(xt, lut, n_seq, n_tok):
    n_sblk = n_seq // SB
    n_blocks = n_tok // TB

    mesh = plsc.VectorSubcoreMesh(core_axis_name="c", subcore_axis_name="s")

    @functools.partial(
        pl.kernel,
        out_type=jax.ShapeDtypeStruct((n_tok, D // 8, n_sblk, 8, SB), jnp.float32),
        mesh=mesh,
        scratch_types=[
            pltpu.VMEM((TB, SB), jnp.int32),
            pltpu.VMEM((TB, SB), jnp.int32),
            pltpu.VMEM((TB, SB, D), jnp.float32),
            pltpu.VMEM((TB, SB, D), jnp.float32),
            pltpu.VMEM((TB, D // 8, 8, SB + 1), jnp.float32),
            pltpu.VMEM((TB, D // 8, 8, SB + 1), jnp.float32),
            pltpu.SemaphoreType.DMA,
            pltpu.SemaphoreType.DMA,
            pltpu.SemaphoreType.DMA,
            pltpu.SemaphoreType.DMA,
        ],
        compiler_params=pltpu.CompilerParams(
            use_tc_tiling_on_sc=False, needs_layout_passes=False
        ),
    )
    def k(xt_hbm, lut_hbm, out_hbm, i0, i1, g0, g1, t0, t1, gs0, gs1, os0, os1):
        idx = (i0, i1)
        gbuf = (g0, g1)
        tbuf = (t0, t1)
        gsem = (gs0, gs1)
        osem = (os0, os1)

        w = lax.axis_index("s") * _NC + lax.axis_index("c")
        s0 = w * SB

        def fire(blk, b):
            pltpu.sync_copy(
                xt_hbm.at[pl.ds(blk * TB, TB), pl.ds(s0, SB)], idx[b]
            )
            for kk in range(TB):
                pltpu.async_copy(
                    lut_hbm.at[idx[b].at[kk]], gbuf[b].at[kk], gsem[b]
                )

        def drain_gather(b):
            for kk in range(TB):
                pltpu.make_async_copy(
                    lut_hbm.at[idx[b].at[kk]], gbuf[b].at[kk], gsem[b]
                ).wait()

        def wait_out(blk, b):
            for kk in range(TB):
                pltpu.make_async_copy(
                    tbuf[b].at[kk, :, :, pl.ds(0, SB)], out_hbm.at[blk * TB + kk, :, w], osem[b]
                ).wait()

        dvecs = []
        for j in range(D // 16):
            dv = lax.iota(jnp.int32, 16) + 16 * j
            dvecs.append((dv // 8, dv % 8))

        fire(0, 0)

        def body(b2, carry):
            for b in range(2):
                blk = b2 * 2 + b
                nb = 1 - b

                @pl.when(blk + 1 < n_blocks)
                def _():
                    fire(blk + 1, nb)

                drain_gather(b)

                @pl.when(blk >= 2)
                def _():
                    wait_out(blk - 2, b)

                for kk in range(TB):
                    src = gbuf[b].at[kk]
                    dst = tbuf[b].at[kk]

                    @plsc.parallel_loop(0, SB, 1, unroll=8)
                    def _(s, src=src, dst=dst):
                        svec = jnp.full((16,), 0, jnp.int32) + s
                        for j in range(D // 16):
                            v = src[s, pl.ds(16 * j, 16)]
                            plsc.store_scatter(
                                dst, [dvecs[j][0], dvecs[j][1], svec], v * SCALE
                            )
                    pltpu.async_copy(
                        tbuf[b].at[kk, :, :, pl.ds(0, SB)], out_hbm.at[blk * TB + kk, :, w], osem[b]
                    )
            return carry

        lax.fori_loop(0, n_blocks // 2, body, 0)
        wait_out(n_blocks - 2, 0)
        wait_out(n_blocks - 1, 1)

    return k(xt, lut)


def kernel(x, lut):
    n_seq, n_tok = x.shape
    xt = x.T
    out5 = _lookup_t_major(xt, lut, n_seq, n_tok)
    out = out5.transpose(2, 4, 0, 1, 3).reshape(n_seq, n_tok, D)
    return out


# speedup vs baseline: 1.4068x; 1.4068x over previous
import functools
import math

import jax
import jax.numpy as jnp
from jax import lax
from jax.experimental import pallas as pl
from jax.experimental.pallas import tpu as pltpu
from jax.experimental.pallas import tpu_sc as plsc

D = 64
SCALE = math.sqrt(D)

_NC = 2
_NS = 16
_NW = _NC * _NS

SB = 128
TB = 2


@functools.partial(jax.jit, static_argnames=("n_seq", "n_tok"))
def _lookup_t_major(xt, lut, n_seq, n_tok):
    n_sblk = n_seq // SB
    n_blocks = n_tok // TB

    mesh = plsc.VectorSubcoreMesh(core_axis_name="c", subcore_axis_name="s")

    @functools.partial(
        pl.kernel,
        out_type=jax.ShapeDtypeStruct((n_tok, D // 8, n_sblk, 8, SB), jnp.float32),
        mesh=mesh,
        scratch_types=[
            pltpu.VMEM((TB, SB), jnp.int32),
            pltpu.VMEM((TB, SB), jnp.int32),
            pltpu.VMEM((TB, SB, D), jnp.float32),
            pltpu.VMEM((TB, SB, D), jnp.float32),
            pltpu.VMEM((TB, D // 8, 8, SB + 1), jnp.float32),
            pltpu.VMEM((TB, D // 8, 8, SB + 1), jnp.float32),
            pltpu.SemaphoreType.DMA,
            pltpu.SemaphoreType.DMA,
            pltpu.SemaphoreType.DMA,
            pltpu.SemaphoreType.DMA,
        ],
        compiler_params=pltpu.CompilerParams(
            use_tc_tiling_on_sc=False, needs_layout_passes=False
        ),
    )
    def k(xt_hbm, lut_hbm, out_hbm, i0, i1, g0, g1, t0, t1, gs0, gs1, os0, os1):
        idx = (i0, i1)
        gbuf = (g0, g1)
        tbuf = (t0, t1)
        gsem = (gs0, gs1)
        osem = (os0, os1)

        w = lax.axis_index("s") * _NC + lax.axis_index("c")
        s0 = w * SB

        def fire(blk, b):
            pltpu.sync_copy(
                xt_hbm.at[pl.ds(blk * TB, TB), pl.ds(s0, SB)], idx[b]
            )
            for kk in range(TB):
                pltpu.async_copy(
                    lut_hbm.at[idx[b].at[kk]], gbuf[b].at[kk], gsem[b]
                )

        def drain_gather(b):
            for kk in range(TB):
                pltpu.make_async_copy(
                    lut_hbm.at[idx[b].at[kk]], gbuf[b].at[kk], gsem[b]
                ).wait()

        def wait_out(blk, b):
            for kk in range(TB):
                pltpu.make_async_copy(
                    tbuf[b].at[kk, :, :, pl.ds(0, SB)], out_hbm.at[blk * TB + kk, :, w], osem[b]
                ).wait()

        dvecs = []
        for j in range(D // 16):
            dv = lax.iota(jnp.int32, 16) + 16 * j
            dvecs.append((dv // 8, dv % 8))

        fire(0, 0)

        def body(b2, carry):
            for b in range(2):
                blk = b2 * 2 + b
                nb = 1 - b

                @pl.when(blk + 1 < n_blocks)
                def _():
                    fire(blk + 1, nb)

                drain_gather(b)

                @pl.when(blk >= 2)
                def _():
                    wait_out(blk - 2, b)

                for kk in range(TB):
                    src = gbuf[b].at[kk]
                    dst = tbuf[b].at[kk]

                    @plsc.parallel_loop(0, SB, 1, unroll=8)
                    def _(s, src=src, dst=dst):
                        svec = jnp.full((16,), 0, jnp.int32) + s
                        for j in range(D // 16):
                            v = src[s, pl.ds(16 * j, 16)]
                            plsc.store_scatter(
                                dst, [dvecs[j][0], dvecs[j][1], svec], v * SCALE
                            )
                    pltpu.async_copy(
                        tbuf[b].at[kk, :, :, pl.ds(0, SB)], out_hbm.at[blk * TB + kk, :, w], osem[b]
                    )
            return carry

        lax.fori_loop(0, n_blocks // 2, body, 0)
        wait_out(n_blocks - 2, 0)
        wait_out(n_blocks - 1, 1)

    return k(xt, lut)


@jax.jit
def _repack_table(lutT, tail_lin):
    n_d, n_vocab = lutT.shape
    n_full = n_vocab // SB
    n_blocks = n_full

    mesh = plsc.VectorSubcoreMesh(core_axis_name="c", subcore_axis_name="s")

    @functools.partial(
        pl.kernel,
        out_type=jax.ShapeDtypeStruct((n_vocab // 2, 2 * n_d), jnp.float32),
        mesh=mesh,
        scratch_types=[
            pltpu.VMEM((n_d, SB), jnp.float32),
            pltpu.VMEM((n_d, SB), jnp.float32),
            pltpu.VMEM((n_d, 2 * n_d + 2), jnp.float32),
            pltpu.VMEM((n_d, 2 * n_d + 2), jnp.float32),
            pltpu.SemaphoreType.DMA,
            pltpu.SemaphoreType.DMA,
            pltpu.SemaphoreType.DMA,
            pltpu.SemaphoreType.DMA,
        ],
        compiler_params=pltpu.CompilerParams(
            use_tc_tiling_on_sc=True, needs_layout_passes=False
        ),
    )
    def k(lutT_hbm, tail_hbm, p_hbm, in0, in1, st0, st1, gs0, gs1, os0, os1):
        ibuf = (in0, in1)
        stag = (st0, st1)
        gsem = (gs0, gs1)
        osem = (os0, os1)
        M = 2 * n_d + 2

        w = lax.axis_index("s") * _NC + lax.axis_index("c")
        n_iter = (n_blocks + _NW - 1) // _NW

        rvecs = []
        cvecs = []
        for j in range(SB // 16):
            il = lax.iota(jnp.int32, 16) + 16 * j
            rvecs.append(il // 2)
            cvecs.append((il % 2) * (M // 2))

        def fire(b, s):
            pltpu.async_copy(
                lutT_hbm.at[:, pl.ds(b * SB, SB)], ibuf[s], gsem[s]
            )

        def wait_read(b, s):
            pltpu.make_async_copy(
                lutT_hbm.at[:, pl.ds(b * SB, SB)], ibuf[s], gsem[s]
            ).wait()

        def fire_out(b, s):
            pltpu.async_copy(
                stag[s].at[:, pl.ds(0, 2 * n_d)],
                p_hbm.at[pl.ds(b * n_d, n_d)],
                osem[s],
            )

        def wait_out(b, s):
            pltpu.make_async_copy(
                stag[s].at[:, pl.ds(0, 2 * n_d)],
                p_hbm.at[pl.ds(b * n_d, n_d)],
                osem[s],
            ).wait()

        @pl.when(w == 0)
        def _():
            pltpu.sync_copy(
                tail_hbm, p_hbm.at[pl.ds(n_full * n_d, n_vocab // 2 - n_full * n_d)]
            )

        fire(w, 0)

        def body(n2, carry):
            for s in range(2):
                n = n2 * 2 + s
                ns = 1 - s
                b = n * _NW + w

                @pl.when(b < n_blocks)
                def _():
                    nxt = b + _NW

                    @pl.when(nxt < n_blocks)
                    def _():
                        fire(nxt, ns)

                    wait_read(b, s)

                    @pl.when(n >= 2)
                    def _():
                        wait_out(b - 2 * _NW, s)

                    sref = ibuf[s]
                    dref = stag[s]

                    @plsc.parallel_loop(0, n_d, 1, unroll=8)
                    def _(d, sref=sref, dref=dref):
                        dvec = jnp.full((16,), 0, jnp.int32) + d
                        for j in range(SB // 16):
                            v = sref[d, pl.ds(16 * j, 16)]
                            plsc.store_scatter(
                                dref, [rvecs[j], cvecs[j] + dvec], v
                            )

                    fire_out(b, s)
            return carry

        n_iter2 = (n_iter + 1) // 2
        lax.fori_loop(0, n_iter2, body, 0)

        def final_wait(n, s):
            b = n * _NW + w

            @pl.when(b < n_blocks)
            def _():
                wait_out(b, s)

        final_wait(n_iter - 2, (n_iter - 2) % 2)
        final_wait(n_iter - 1, (n_iter - 1) % 2)

    return k(lutT, tail_lin)


def kernel(x, lut):
    n_seq, n_tok = x.shape
    xt = x.T
    tail_rows = lut.shape[0] - (lut.shape[0] // SB) * SB
    tail_lin = lax.slice(lut, (lut.shape[0] - tail_rows, 0), lut.shape)
    tail_lin = tail_lin.reshape(tail_rows // 2, 2 * D)
    packed = _repack_table(lut.T, tail_lin)
    lin = packed.reshape(lut.shape[0], D)
    out5 = _lookup_t_major(xt, lin, n_seq, n_tok)
    out = out5.transpose(2, 4, 0, 1, 3).reshape(n_seq, n_tok, D)
    return out
